# initial kernel scaffold (unmeasured)
import jax
import jax.numpy as jnp
from jax import lax
from jax.experimental import pallas as pl
from jax.experimental.pallas import tpu as pltpu

N_Z = 4
S = 1024
D = 2048
DC = 128
H = 16
DH = 128
DR = 32


def _gather_kv(x2, Wdkv, Wuk, Wuv):

    def body(x_ref, wdkv_ref, wuk_ref, wuv_ref, k_ref, v_ref,
             c_all, uk_all, uv_all, kacc, vacc, send_sems, recv_sems):
        my_x = lax.axis_index("x")
        my_y = lax.axis_index("y")
        my_z = lax.axis_index("z")
        right = lax.rem(my_z + 1, N_Z)
        left = lax.rem(my_z + N_Z - 1, N_Z)

        barrier = pltpu.get_barrier_semaphore()
        for nbr in (left, right):
            pl.semaphore_signal(
                barrier, inc=1,
                device_id=(my_x, my_y, nbr),
                device_id_type=pl.DeviceIdType.MESH,
            )
        pl.semaphore_wait(barrier, 2)

        c0 = jnp.dot(x_ref[...], wdkv_ref[...],
                     preferred_element_type=jnp.float32)
        c_all[0] = c0.astype(jnp.bfloat16)
        uk_all[0] = wuk_ref[...]
        uv_all[0] = wuv_ref[...]

        kacc[...] = jnp.dot(c_all[0], uk_all[0],
                            preferred_element_type=jnp.float32)
        vacc[...] = jnp.dot(c_all[0], uv_all[0],
                            preferred_element_type=jnp.float32)

        for h in range(N_Z - 1):
            rdmas = []
            for t, buf in enumerate((c_all, uk_all, uv_all)):
                r = pltpu.make_async_remote_copy(
                    src_ref=buf.at[h],
                    dst_ref=buf.at[h + 1],
                    send_sem=send_sems.at[t, h],
                    recv_sem=recv_sems.at[t, h],
                    device_id=(my_x, my_y, right),
                    device_id_type=pl.DeviceIdType.MESH,
                )
                r.start()
                rdmas.append(r)
            for r in rdmas:
                r.wait()
            kacc[...] += jnp.dot(c_all[h + 1], uk_all[h + 1],
                                 preferred_element_type=jnp.float32)
            vacc[...] += jnp.dot(c_all[h + 1], uv_all[h + 1],
                                 preferred_element_type=jnp.float32)

        k_ref[...] = kacc[...].astype(jnp.bfloat16)
        v_ref[...] = vacc[...].astype(jnp.bfloat16)

    return pl.pallas_call(
        body,
        out_shape=(
            jax.ShapeDtypeStruct((S, D), jnp.bfloat16),
            jax.ShapeDtypeStruct((S, D), jnp.bfloat16),
        ),
        in_specs=[pl.BlockSpec(memory_space=pltpu.VMEM)] * 4,
        out_specs=(
            pl.BlockSpec(memory_space=pltpu.VMEM),
            pl.BlockSpec(memory_space=pltpu.VMEM),
        ),
        scratch_shapes=[
            pltpu.VMEM((N_Z, S, DC), jnp.bfloat16),
            pltpu.VMEM((N_Z, DC, D), jnp.bfloat16),
            pltpu.VMEM((N_Z, DC, D), jnp.bfloat16),
            pltpu.VMEM((S, D), jnp.float32),
            pltpu.VMEM((S, D), jnp.float32),
            pltpu.SemaphoreType.DMA((3, N_Z - 1)),
            pltpu.SemaphoreType.DMA((3, N_Z - 1)),
        ],
        compiler_params=pltpu.CompilerParams(collective_id=0),
    )(x2, Wdkv, Wuk, Wuv)


def _attention(x2, Wq, Wqr, Wkr, Wo, K, V):
    scale = (DH + DR) ** -0.5

    def body(x_ref, wq_ref, wqr_ref, wkr_ref, wo_ref, k_ref, v_ref, out_ref):
        h = pl.program_id(0)
        xb = x_ref[...]
        q = jnp.dot(xb, wq_ref[...],
                    preferred_element_type=jnp.float32).astype(jnp.bfloat16)
        qr = jnp.dot(xb, wqr_ref[...],
                     preferred_element_type=jnp.float32).astype(jnp.bfloat16)
        kr = jnp.dot(xb, wkr_ref[...],
                     preferred_element_type=jnp.float32).astype(jnp.bfloat16)
        s = lax.dot_general(
            q, k_ref[...], (((1,), (1,)), ((), ())),
            preferred_element_type=jnp.float32,
        )
        s += lax.dot_general(
            qr, kr, (((1,), (1,)), ((), ())),
            preferred_element_type=jnp.float32,
        )
        s *= scale
        m = jnp.max(s, axis=-1, keepdims=True)
        p = jnp.exp(s - m)
        p /= jnp.sum(p, axis=-1, keepdims=True)
        o = jnp.dot(p.astype(jnp.bfloat16), v_ref[...],
                    preferred_element_type=jnp.float32)
        contrib = jnp.dot(o.astype(jnp.bfloat16), wo_ref[...],
                          preferred_element_type=jnp.float32)

        @pl.when(h == 0)
        def _():
            out_ref[...] = contrib

        @pl.when(h != 0)
        def _():
            out_ref[...] += contrib

    return pl.pallas_call(
        body,
        grid=(H,),
        out_shape=jax.ShapeDtypeStruct((S, D), jnp.float32),
        in_specs=[
            pl.BlockSpec((S, D), lambda h: (0, 0)),
            pl.BlockSpec((D, DH), lambda h: (0, h)),
            pl.BlockSpec((D, DR), lambda h: (0, h)),
            pl.BlockSpec((D, DR), lambda h: (0, 0)),
            pl.BlockSpec((DH, D), lambda h: (h, 0)),
            pl.BlockSpec((S, DH), lambda h: (0, h)),
            pl.BlockSpec((S, DH), lambda h: (0, h)),
        ],
        out_specs=pl.BlockSpec((S, D), lambda h: (0, 0)),
    )(x2, Wq, Wqr, Wkr, Wo, K, V)


def kernel(x, Wdkv, Wuk, Wuv, Wq, Wqr, Wkr, Wo):
    x2 = x[0].astype(jnp.bfloat16)
    K, V = _gather_kv(
        x2,
        Wdkv.astype(jnp.bfloat16),
        Wuk.astype(jnp.bfloat16),
        Wuv.astype(jnp.bfloat16),
    )
    out = _attention(
        x2,
        Wq.astype(jnp.bfloat16),
        Wqr.astype(jnp.bfloat16),
        Wkr.astype(jnp.bfloat16),
        Wo.astype(jnp.bfloat16),
        K, V,
    )
    return out[None]


# baseline (device time: 209500 ns/iter reference)
import jax
import jax.numpy as jnp
from jax import lax
from jax.experimental import pallas as pl
from jax.experimental.pallas import tpu as pltpu

N_Z = 4
S = 1024
D = 2048
DC = 128
H = 16
DH = 128
DR = 32


def _gather_kv(x2, Wdkv, Wuk, Wuv):

    def body(x_ref, wdkv_ref, wuk_ref, wuv_ref, k_ref, v_ref,
             c_all, uk_all, uv_all, kacc, vacc, send_sems, recv_sems):
        my_x = lax.axis_index("x")
        my_y = lax.axis_index("y")
        my_z = lax.axis_index("z")
        right = lax.rem(my_z + 1, N_Z)
        left = lax.rem(my_z + N_Z - 1, N_Z)

        barrier = pltpu.get_barrier_semaphore()
        for nbr in (left, right):
            pl.semaphore_signal(
                barrier, inc=1,
                device_id=(my_x, my_y, nbr),
                device_id_type=pl.DeviceIdType.MESH,
            )
        pl.semaphore_wait(barrier, 2)

        c0 = jnp.dot(x_ref[...], wdkv_ref[...],
                     preferred_element_type=jnp.float32)
        c_all[0] = c0.astype(jnp.bfloat16)
        uk_all[0] = wuk_ref[...]
        uv_all[0] = wuv_ref[...]

        kacc[...] = jnp.dot(c_all[0], uk_all[0],
                            preferred_element_type=jnp.float32)
        vacc[...] = jnp.dot(c_all[0], uv_all[0],
                            preferred_element_type=jnp.float32)

        for h in range(N_Z - 1):
            rdmas = []
            for t, buf in enumerate((c_all, uk_all, uv_all)):
                r = pltpu.make_async_remote_copy(
                    src_ref=buf.at[h],
                    dst_ref=buf.at[h + 1],
                    send_sem=send_sems.at[t, h],
                    recv_sem=recv_sems.at[t, h],
                    device_id=(my_x, my_y, right),
                    device_id_type=pl.DeviceIdType.MESH,
                )
                r.start()
                rdmas.append(r)
            for r in rdmas:
                r.wait()
            kacc[...] += jnp.dot(c_all[h + 1], uk_all[h + 1],
                                 preferred_element_type=jnp.float32)
            vacc[...] += jnp.dot(c_all[h + 1], uv_all[h + 1],
                                 preferred_element_type=jnp.float32)

        k_ref[...] = kacc[...].astype(jnp.bfloat16)
        v_ref[...] = vacc[...].astype(jnp.bfloat16)

    return pl.pallas_call(
        body,
        out_shape=(
            jax.ShapeDtypeStruct((S, D), jnp.bfloat16),
            jax.ShapeDtypeStruct((S, D), jnp.bfloat16),
        ),
        in_specs=[pl.BlockSpec(memory_space=pltpu.VMEM)] * 4,
        out_specs=(
            pl.BlockSpec(memory_space=pltpu.VMEM),
            pl.BlockSpec(memory_space=pltpu.VMEM),
        ),
        scratch_shapes=[
            pltpu.VMEM((N_Z, S, DC), jnp.bfloat16),
            pltpu.VMEM((N_Z, DC, D), jnp.bfloat16),
            pltpu.VMEM((N_Z, DC, D), jnp.bfloat16),
            pltpu.VMEM((S, D), jnp.float32),
            pltpu.VMEM((S, D), jnp.float32),
            pltpu.SemaphoreType.DMA((3, N_Z - 1)),
            pltpu.SemaphoreType.DMA((3, N_Z - 1)),
        ],
        compiler_params=pltpu.CompilerParams(
            collective_id=0,
            vmem_limit_bytes=100 * 1024 * 1024,
        ),
    )(x2, Wdkv, Wuk, Wuv)


G = 4
GDH = G * DH
GDR = G * DR


def _attention(x2, Wq, Wqr, Wkr, Wo, K, V):
    scale = (DH + DR) ** -0.5

    def body(x_ref, wq_ref, wqr_ref, wkr_ref, wo_ref, k_ref, v_ref, out_ref):
        g = pl.program_id(0)
        xb = x_ref[...]
        q4 = jnp.dot(xb, wq_ref[...],
                     preferred_element_type=jnp.float32).astype(jnp.bfloat16)
        qr4 = jnp.dot(xb, wqr_ref[...],
                      preferred_element_type=jnp.float32).astype(jnp.bfloat16)
        kr = jnp.dot(xb, wkr_ref[...],
                     preferred_element_type=jnp.float32).astype(jnp.bfloat16)
        k4 = k_ref[...]
        v4 = v_ref[...]
        wo4 = wo_ref[...]

        @pl.when(g == 0)
        def _():
            out_ref[...] = jnp.zeros((S, D), jnp.float32)

        for i in range(G):
            qh = q4[:, i * DH:(i + 1) * DH]
            qrh = qr4[:, i * DR:(i + 1) * DR]
            kh = k4[:, i * DH:(i + 1) * DH]
            vh = v4[:, i * DH:(i + 1) * DH]
            s = lax.dot_general(
                qh, kh, (((1,), (1,)), ((), ())),
                preferred_element_type=jnp.float32,
            )
            s += lax.dot_general(
                qrh, kr, (((1,), (1,)), ((), ())),
                preferred_element_type=jnp.float32,
            )
            s *= scale
            m = jnp.max(s, axis=-1, keepdims=True)
            p = jnp.exp(s - m)
            p /= jnp.sum(p, axis=-1, keepdims=True)
            o = jnp.dot(p.astype(jnp.bfloat16), vh,
                        preferred_element_type=jnp.float32)
            out_ref[...] += jnp.dot(o.astype(jnp.bfloat16),
                                    wo4[i * DH:(i + 1) * DH, :],
                                    preferred_element_type=jnp.float32)

    return pl.pallas_call(
        body,
        grid=(H // G,),
        out_shape=jax.ShapeDtypeStruct((S, D), jnp.float32),
        in_specs=[
            pl.BlockSpec((S, D), lambda g: (0, 0)),
            pl.BlockSpec((D, GDH), lambda g: (0, g)),
            pl.BlockSpec((D, GDR), lambda g: (0, g)),
            pl.BlockSpec((D, DR), lambda g: (0, 0)),
            pl.BlockSpec((GDH, D), lambda g: (g, 0)),
            pl.BlockSpec((S, GDH), lambda g: (0, g)),
            pl.BlockSpec((S, GDH), lambda g: (0, g)),
        ],
        out_specs=pl.BlockSpec((S, D), lambda g: (0, 0)),
        compiler_params=pltpu.CompilerParams(
            vmem_limit_bytes=100 * 1024 * 1024,
        ),
    )(x2, Wq, Wqr, Wkr, Wo, K, V)


def kernel(x, Wdkv, Wuk, Wuv, Wq, Wqr, Wkr, Wo):
    x2 = x[0].astype(jnp.bfloat16)
    K, V = _gather_kv(
        x2,
        Wdkv.astype(jnp.bfloat16),
        Wuk.astype(jnp.bfloat16),
        Wuv.astype(jnp.bfloat16),
    )
    out = _attention(
        x2,
        Wq.astype(jnp.bfloat16),
        Wqr.astype(jnp.bfloat16),
        Wkr.astype(jnp.bfloat16),
        Wo.astype(jnp.bfloat16),
        K, V,
    )
    return out[None]


# device time: 138652 ns/iter; 1.5110x vs baseline; 1.5110x over previous
import jax
import jax.numpy as jnp
from jax import lax
from jax.experimental import pallas as pl
from jax.experimental.pallas import tpu as pltpu

N_Z = 4
S = 1024
D = 2048
DC = 128
H = 16
DH = 128
DR = 32


def _gather_kv(x2, Wdkv, Wuk, Wuv):

    def body(x_ref, wdkv_ref, wuk_ref, wuv_ref, k_ref, v_ref,
             c_all, uk_all, uv_all, kacc, vacc, send_sems, recv_sems):
        my_x = lax.axis_index("x")
        my_y = lax.axis_index("y")
        my_z = lax.axis_index("z")
        right = lax.rem(my_z + 1, N_Z)
        left = lax.rem(my_z + N_Z - 1, N_Z)

        barrier = pltpu.get_barrier_semaphore()
        for nbr in (left, right):
            pl.semaphore_signal(
                barrier, inc=1,
                device_id=(my_x, my_y, nbr),
                device_id_type=pl.DeviceIdType.MESH,
            )
        pl.semaphore_wait(barrier, 2)

        xb = x_ref[...].astype(jnp.bfloat16)
        c0 = jnp.dot(xb, wdkv_ref[...].astype(jnp.bfloat16),
                     preferred_element_type=jnp.float32)
        c_all[0] = c0.astype(jnp.bfloat16)
        uk_all[0] = wuk_ref[...].astype(jnp.bfloat16)
        uv_all[0] = wuv_ref[...].astype(jnp.bfloat16)

        def hop_rdmas(h):
            return [
                pltpu.make_async_remote_copy(
                    src_ref=buf.at[h],
                    dst_ref=buf.at[h + 1],
                    send_sem=send_sems.at[t, h],
                    recv_sem=recv_sems.at[t, h],
                    device_id=(my_x, my_y, right),
                    device_id_type=pl.DeviceIdType.MESH,
                )
                for t, buf in enumerate((c_all, uk_all, uv_all))
            ]

        all_rdmas = []
        rdmas = hop_rdmas(0)
        for r in rdmas:
            r.start()
        all_rdmas += rdmas

        kacc[...] = jnp.dot(c_all[0], uk_all[0],
                            preferred_element_type=jnp.float32)
        vacc[...] = jnp.dot(c_all[0], uv_all[0],
                            preferred_element_type=jnp.float32)

        for h in range(N_Z - 1):
            for r in all_rdmas[3 * h:3 * h + 3]:
                r.wait_recv()
            if h < N_Z - 2:
                rdmas = hop_rdmas(h + 1)
                for r in rdmas:
                    r.start()
                all_rdmas += rdmas
            kacc[...] += jnp.dot(c_all[h + 1], uk_all[h + 1],
                                 preferred_element_type=jnp.float32)
            vacc[...] += jnp.dot(c_all[h + 1], uv_all[h + 1],
                                 preferred_element_type=jnp.float32)

        k_ref[...] = kacc[...].astype(jnp.bfloat16)
        v_ref[...] = vacc[...].astype(jnp.bfloat16)
        for r in all_rdmas:
            r.wait_send()

    return pl.pallas_call(
        body,
        out_shape=(
            jax.ShapeDtypeStruct((S, D), jnp.bfloat16),
            jax.ShapeDtypeStruct((S, D), jnp.bfloat16),
        ),
        in_specs=[pl.BlockSpec(memory_space=pltpu.VMEM)] * 4,
        out_specs=(
            pl.BlockSpec(memory_space=pltpu.VMEM),
            pl.BlockSpec(memory_space=pltpu.VMEM),
        ),
        scratch_shapes=[
            pltpu.VMEM((N_Z, S, DC), jnp.bfloat16),
            pltpu.VMEM((N_Z, DC, D), jnp.bfloat16),
            pltpu.VMEM((N_Z, DC, D), jnp.bfloat16),
            pltpu.VMEM((S, D), jnp.float32),
            pltpu.VMEM((S, D), jnp.float32),
            pltpu.SemaphoreType.DMA((3, N_Z - 1)),
            pltpu.SemaphoreType.DMA((3, N_Z - 1)),
        ],
        compiler_params=pltpu.CompilerParams(
            collective_id=0,
            vmem_limit_bytes=100 * 1024 * 1024,
        ),
    )(x2, Wdkv, Wuk, Wuv)


G = 4
GDH = G * DH
GDR = G * DR


def _attention(x2, Wq, Wqr, Wkr, Wo, K, V):
    scale = (DH + DR) ** -0.5

    def body(x_ref, wq_ref, wqr_ref, wkr_ref, wo_ref, k_ref, v_ref, out_ref):
        g = pl.program_id(0)
        xb = x_ref[...]
        wq4 = wq_ref[...].astype(jnp.bfloat16)
        wqr4 = wqr_ref[...].astype(jnp.bfloat16)
        wkrb = wkr_ref[...].astype(jnp.bfloat16)
        wo4 = wo_ref[...].astype(jnp.bfloat16)
        q4 = jnp.dot(xb, wq4,
                     preferred_element_type=jnp.float32).astype(jnp.bfloat16)
        qr4 = jnp.dot(xb, wqr4,
                      preferred_element_type=jnp.float32).astype(jnp.bfloat16)
        kr = jnp.dot(xb, wkrb,
                     preferred_element_type=jnp.float32).astype(jnp.bfloat16)
        k4 = k_ref[...]
        v4 = v_ref[...]

        o_parts = []
        for i in range(G):
            qaug = jnp.concatenate(
                [q4[:, i * DH:(i + 1) * DH], qr4[:, i * DR:(i + 1) * DR]],
                axis=1)
            kaug = jnp.concatenate(
                [k4[:, i * DH:(i + 1) * DH], kr], axis=1)
            s = lax.dot_general(
                qaug, kaug, (((1,), (1,)), ((), ())),
                preferred_element_type=jnp.float32,
            )
            p = jnp.exp(s * scale)
            r = 1.0 / jnp.sum(p, axis=-1, keepdims=True)
            o = jnp.dot(p.astype(jnp.bfloat16), v4[:, i * DH:(i + 1) * DH],
                        preferred_element_type=jnp.float32)
            o_parts.append((o * r).astype(jnp.bfloat16))
        o4 = jnp.concatenate(o_parts, axis=1)
        contrib = jnp.dot(o4, wo4, preferred_element_type=jnp.float32)

        @pl.when(g == 0)
        def _():
            out_ref[...] = contrib

        @pl.when(g != 0)
        def _():
            out_ref[...] += contrib

    return pl.pallas_call(
        body,
        grid=(H // G,),
        out_shape=jax.ShapeDtypeStruct((S, D), jnp.float32),
        in_specs=[
            pl.BlockSpec((S, D), lambda g: (0, 0)),
            pl.BlockSpec((D, GDH), lambda g: (0, g)),
            pl.BlockSpec((D, GDR), lambda g: (0, g)),
            pl.BlockSpec((D, DR), lambda g: (0, 0)),
            pl.BlockSpec((GDH, D), lambda g: (g, 0)),
            pl.BlockSpec((S, GDH), lambda g: (0, g)),
            pl.BlockSpec((S, GDH), lambda g: (0, g)),
        ],
        out_specs=pl.BlockSpec((S, D), lambda g: (0, 0)),
        compiler_params=pltpu.CompilerParams(
            vmem_limit_bytes=100 * 1024 * 1024,
        ),
    )(x2, Wq, Wqr, Wkr, Wo, K, V)


def kernel(x, Wdkv, Wuk, Wuv, Wq, Wqr, Wkr, Wo):
    x2 = x[0].astype(jnp.bfloat16)
    K, V = _gather_kv(x[0], Wdkv, Wuk, Wuv)
    out = _attention(x2, Wq, Wqr, Wkr, Wo, K, V)
    return out[None]


# device time: 136023 ns/iter; 1.5402x vs baseline; 1.0193x over previous
import jax
import jax.numpy as jnp
from jax import lax
from jax.experimental import pallas as pl
from jax.experimental.pallas import tpu as pltpu

N_Z = 4
S = 1024
D = 2048
DC = 128
H = 16
DH = 128
DR = 32


HC = DC // 2


def _gather_kv(x2, Wdkv, Wuk, Wuv):

    def body(x_ref, wdkv_ref, wuk_ref, wuv_ref, k_ref, v_ref,
             c_own, uk_own, uv_own, c_oth, uk_oth, uv_oth,
             kacc, vacc, z_send, z_recv, y_send, y_recv):
        my_x = lax.axis_index("x")
        my_y = lax.axis_index("y")
        my_z = lax.axis_index("z")
        right = lax.rem(my_z + 1, N_Z)
        left = lax.rem(my_z + N_Z - 1, N_Z)
        my_half = lax.rem(my_y, 2)
        partner = my_y + 1 - 2 * my_half

        barrier = pltpu.get_barrier_semaphore()
        for dev in ((my_x, my_y, left), (my_x, my_y, right),
                    (my_x, partner, my_z)):
            pl.semaphore_signal(
                barrier, inc=1,
                device_id=dev,
                device_id_type=pl.DeviceIdType.MESH,
            )
        pl.semaphore_wait(barrier, 3)

        xb = x_ref[...].astype(jnp.bfloat16)
        c0 = jnp.dot(xb, wdkv_ref[...].astype(jnp.bfloat16),
                     preferred_element_type=jnp.float32).astype(jnp.bfloat16)
        ukb = wuk_ref[...].astype(jnp.bfloat16)
        uvb = wuv_ref[...].astype(jnp.bfloat16)

        @pl.when(my_half == 0)
        def _():
            c_own[0] = c0[:, :HC]
            uk_own[0] = ukb[:HC, :]
            uv_own[0] = uvb[:HC, :]

        @pl.when(my_half == 1)
        def _():
            c_own[0] = c0[:, HC:]
            uk_own[0] = ukb[HC:, :]
            uv_own[0] = uvb[HC:, :]

        def rdmas_for(h, bufs_src, bufs_dst, send_sems, recv_sems, dev):
            return [
                pltpu.make_async_remote_copy(
                    src_ref=src.at[h + 1] if src is not dst else src.at[h],
                    dst_ref=dst.at[h + 1],
                    send_sem=send_sems.at[t, h],
                    recv_sem=recv_sems.at[t, h],
                    device_id=dev,
                    device_id_type=pl.DeviceIdType.MESH,
                )
                for t, (src, dst) in enumerate(zip(bufs_src, bufs_dst))
            ]

        own = (c_own, uk_own, uv_own)
        oth = (c_oth, uk_oth, uv_oth)
        z_dev = (my_x, my_y, right)
        y_dev = (my_x, partner, my_z)

        pending = []
        zr = rdmas_for(0, own, own, z_send, z_recv, z_dev)
        for r in zr:
            r.start()
        pending += zr

        kacc[...] = jnp.dot(c0, ukb, preferred_element_type=jnp.float32)
        vacc[...] = jnp.dot(c0, uvb, preferred_element_type=jnp.float32)

        y_rdmas = []
        for h in range(N_Z - 1):
            for r in zr:
                r.wait_recv()
            yr = rdmas_for(h, own, oth, y_send, y_recv, y_dev)
            for r in yr:
                r.start()
            pending += yr
            y_rdmas.append(yr)
            if h < N_Z - 2:
                zr = rdmas_for(h + 1, own, own, z_send, z_recv, z_dev)
                for r in zr:
                    r.start()
                pending += zr
            kacc[...] += jnp.dot(c_own[h + 1], uk_own[h + 1],
                                 preferred_element_type=jnp.float32)
            vacc[...] += jnp.dot(c_own[h + 1], uv_own[h + 1],
                                 preferred_element_type=jnp.float32)

        for h in range(N_Z - 1):
            for r in y_rdmas[h]:
                r.wait_recv()
            kacc[...] += jnp.dot(c_oth[h + 1], uk_oth[h + 1],
                                 preferred_element_type=jnp.float32)
            vacc[...] += jnp.dot(c_oth[h + 1], uv_oth[h + 1],
                                 preferred_element_type=jnp.float32)

        k_ref[...] = kacc[...].astype(jnp.bfloat16)
        v_ref[...] = vacc[...].astype(jnp.bfloat16)
        for r in pending:
            r.wait_send()

    return pl.pallas_call(
        body,
        out_shape=(
            jax.ShapeDtypeStruct((S, D), jnp.bfloat16),
            jax.ShapeDtypeStruct((S, D), jnp.bfloat16),
        ),
        in_specs=[pl.BlockSpec(memory_space=pltpu.VMEM)] * 4,
        out_specs=(
            pl.BlockSpec(memory_space=pltpu.VMEM),
            pl.BlockSpec(memory_space=pltpu.VMEM),
        ),
        scratch_shapes=[
            pltpu.VMEM((N_Z, S, HC), jnp.bfloat16),
            pltpu.VMEM((N_Z, HC, D), jnp.bfloat16),
            pltpu.VMEM((N_Z, HC, D), jnp.bfloat16),
            pltpu.VMEM((N_Z, S, HC), jnp.bfloat16),
            pltpu.VMEM((N_Z, HC, D), jnp.bfloat16),
            pltpu.VMEM((N_Z, HC, D), jnp.bfloat16),
            pltpu.VMEM((S, D), jnp.float32),
            pltpu.VMEM((S, D), jnp.float32),
            pltpu.SemaphoreType.DMA((3, N_Z - 1)),
            pltpu.SemaphoreType.DMA((3, N_Z - 1)),
            pltpu.SemaphoreType.DMA((3, N_Z - 1)),
            pltpu.SemaphoreType.DMA((3, N_Z - 1)),
        ],
        compiler_params=pltpu.CompilerParams(
            collective_id=0,
            vmem_limit_bytes=100 * 1024 * 1024,
        ),
    )(x2, Wdkv, Wuk, Wuv)


G = 4
GDH = G * DH
GDR = G * DR


def _attention(x2, Wq, Wqr, Wkr, Wo, K, V):
    scale = (DH + DR) ** -0.5

    def body(x_ref, wq_ref, wqr_ref, wkr_ref, wo_ref, k_ref, v_ref, out_ref,
             kr_s):
        g = pl.program_id(0)
        xb = x_ref[...]
        wq4 = wq_ref[...].astype(jnp.bfloat16)
        wqr4 = wqr_ref[...].astype(jnp.bfloat16)
        wo4 = wo_ref[...].astype(jnp.bfloat16)
        q4 = (jnp.dot(xb, wq4, preferred_element_type=jnp.float32)
              * scale).astype(jnp.bfloat16)
        qr4 = (jnp.dot(xb, wqr4, preferred_element_type=jnp.float32)
               * scale).astype(jnp.bfloat16)

        @pl.when(g == 0)
        def _():
            wkrb = wkr_ref[...].astype(jnp.bfloat16)
            kr_s[...] = jnp.dot(
                xb, wkrb, preferred_element_type=jnp.float32
            ).astype(jnp.bfloat16)

        kr = kr_s[...]
        k4 = k_ref[...]
        v4 = v_ref[...]

        o_parts = []
        for i in range(G):
            qaug = jnp.concatenate(
                [q4[:, i * DH:(i + 1) * DH], qr4[:, i * DR:(i + 1) * DR]],
                axis=1)
            kaug = jnp.concatenate(
                [k4[:, i * DH:(i + 1) * DH], kr], axis=1)
            s = lax.dot_general(
                qaug, kaug, (((1,), (1,)), ((), ())),
                preferred_element_type=jnp.float32,
            )
            p = jnp.exp(s.astype(jnp.bfloat16))
            r = 1.0 / jnp.sum(p, axis=-1, keepdims=True,
                              dtype=jnp.float32)
            o = jnp.dot(p, v4[:, i * DH:(i + 1) * DH],
                        preferred_element_type=jnp.float32)
            o_parts.append((o * r).astype(jnp.bfloat16))
        o4 = jnp.concatenate(o_parts, axis=1)
        contrib = jnp.dot(o4, wo4, preferred_element_type=jnp.float32)

        @pl.when(g == 0)
        def _():
            out_ref[...] = contrib

        @pl.when(g != 0)
        def _():
            out_ref[...] += contrib

    return pl.pallas_call(
        body,
        grid=(H // G,),
        out_shape=jax.ShapeDtypeStruct((S, D), jnp.float32),
        in_specs=[
            pl.BlockSpec((S, D), lambda g: (0, 0)),
            pl.BlockSpec((D, GDH), lambda g: (0, g)),
            pl.BlockSpec((D, GDR), lambda g: (0, g)),
            pl.BlockSpec((D, DR), lambda g: (0, 0)),
            pl.BlockSpec((GDH, D), lambda g: (g, 0)),
            pl.BlockSpec((S, GDH), lambda g: (0, g)),
            pl.BlockSpec((S, GDH), lambda g: (0, g)),
        ],
        out_specs=pl.BlockSpec((S, D), lambda g: (0, 0)),
        scratch_shapes=[
            pltpu.VMEM((S, DR), jnp.bfloat16),
        ],
        compiler_params=pltpu.CompilerParams(
            vmem_limit_bytes=100 * 1024 * 1024,
        ),
    )(x2, Wq, Wqr, Wkr, Wo, K, V)


def kernel(x, Wdkv, Wuk, Wuv, Wq, Wqr, Wkr, Wo):
    x2 = x[0].astype(jnp.bfloat16)
    K, V = _gather_kv(x[0], Wdkv, Wuk, Wuv)
    out = _attention(x2, Wq, Wqr, Wkr, Wo, K, V)
    return out[None]


# device time: 134431 ns/iter; 1.5584x vs baseline; 1.0118x over previous
import jax
import jax.numpy as jnp
from jax import lax
from jax.experimental import pallas as pl
from jax.experimental.pallas import tpu as pltpu

N_Z = 4
S = 1024
D = 2048
DC = 128
H = 16
DH = 128
DR = 32


HC = DC // 2


def _gather_kv(x2, Wdkv, Wuk, Wuv):

    def body(x_ref, wdkv_ref, wuk_ref, wuv_ref, k_ref, v_ref,
             c_own, uk_own, uv_own, c_oth, uk_oth, uv_oth,
             kacc, vacc, z_send, z_recv, y_send, y_recv):
        my_x = lax.axis_index("x")
        my_y = lax.axis_index("y")
        my_z = lax.axis_index("z")
        right = lax.rem(my_z + 1, N_Z)
        left = lax.rem(my_z + N_Z - 1, N_Z)
        my_half = lax.rem(my_y, 2)
        partner = my_y + 1 - 2 * my_half

        barrier = pltpu.get_barrier_semaphore()
        for dev in ((my_x, my_y, left), (my_x, my_y, right),
                    (my_x, partner, my_z)):
            pl.semaphore_signal(
                barrier, inc=1,
                device_id=dev,
                device_id_type=pl.DeviceIdType.MESH,
            )
        pl.semaphore_wait(barrier, 3)

        xb = x_ref[...].astype(jnp.bfloat16)
        c0 = jnp.dot(xb, wdkv_ref[...].astype(jnp.bfloat16),
                     preferred_element_type=jnp.float32).astype(jnp.bfloat16)
        ukb = wuk_ref[...].astype(jnp.bfloat16)
        uvb = wuv_ref[...].astype(jnp.bfloat16)

        @pl.when(my_half == 0)
        def _():
            c_own[0] = c0[:, :HC]
            uk_own[0] = ukb[:HC, :]
            uv_own[0] = uvb[:HC, :]

        @pl.when(my_half == 1)
        def _():
            c_own[0] = c0[:, HC:]
            uk_own[0] = ukb[HC:, :]
            uv_own[0] = uvb[HC:, :]

        def rdmas_for(h, bufs_src, bufs_dst, send_sems, recv_sems, dev):
            return [
                pltpu.make_async_remote_copy(
                    src_ref=src.at[h + 1] if src is not dst else src.at[h],
                    dst_ref=dst.at[h + 1],
                    send_sem=send_sems.at[t, h],
                    recv_sem=recv_sems.at[t, h],
                    device_id=dev,
                    device_id_type=pl.DeviceIdType.MESH,
                )
                for t, (src, dst) in enumerate(zip(bufs_src, bufs_dst))
            ]

        own = (c_own, uk_own, uv_own)
        oth = (c_oth, uk_oth, uv_oth)
        z_dev = (my_x, my_y, right)
        y_dev = (my_x, partner, my_z)

        pending = []
        zr = rdmas_for(0, own, own, z_send, z_recv, z_dev)
        for r in zr:
            r.start()
        pending += zr

        kacc[...] = jnp.dot(c0, ukb, preferred_element_type=jnp.float32)
        vacc[...] = jnp.dot(c0, uvb, preferred_element_type=jnp.float32)

        y_rdmas = []
        for h in range(N_Z - 1):
            for r in zr:
                r.wait_recv()
            yr = rdmas_for(h, own, oth, y_send, y_recv, y_dev)
            for r in yr:
                r.start()
            pending += yr
            y_rdmas.append(yr)
            if h < N_Z - 2:
                zr = rdmas_for(h + 1, own, own, z_send, z_recv, z_dev)
                for r in zr:
                    r.start()
                pending += zr
            kacc[...] += jnp.dot(c_own[h + 1], uk_own[h + 1],
                                 preferred_element_type=jnp.float32)
            vacc[...] += jnp.dot(c_own[h + 1], uv_own[h + 1],
                                 preferred_element_type=jnp.float32)

        for h in range(N_Z - 1):
            for r in y_rdmas[h]:
                r.wait_recv()
            kacc[...] += jnp.dot(c_oth[h + 1], uk_oth[h + 1],
                                 preferred_element_type=jnp.float32)
            vacc[...] += jnp.dot(c_oth[h + 1], uv_oth[h + 1],
                                 preferred_element_type=jnp.float32)

        k_ref[...] = kacc[...].astype(jnp.bfloat16)
        v_ref[...] = vacc[...].astype(jnp.bfloat16)
        for r in pending:
            r.wait_send()

    return pl.pallas_call(
        body,
        out_shape=(
            jax.ShapeDtypeStruct((S, D), jnp.bfloat16),
            jax.ShapeDtypeStruct((S, D), jnp.bfloat16),
        ),
        in_specs=[pl.BlockSpec(memory_space=pltpu.VMEM)] * 4,
        out_specs=(
            pl.BlockSpec(memory_space=pltpu.VMEM),
            pl.BlockSpec(memory_space=pltpu.VMEM),
        ),
        scratch_shapes=[
            pltpu.VMEM((N_Z, S, HC), jnp.bfloat16),
            pltpu.VMEM((N_Z, HC, D), jnp.bfloat16),
            pltpu.VMEM((N_Z, HC, D), jnp.bfloat16),
            pltpu.VMEM((N_Z, S, HC), jnp.bfloat16),
            pltpu.VMEM((N_Z, HC, D), jnp.bfloat16),
            pltpu.VMEM((N_Z, HC, D), jnp.bfloat16),
            pltpu.VMEM((S, D), jnp.float32),
            pltpu.VMEM((S, D), jnp.float32),
            pltpu.SemaphoreType.DMA((3, N_Z - 1)),
            pltpu.SemaphoreType.DMA((3, N_Z - 1)),
            pltpu.SemaphoreType.DMA((3, N_Z - 1)),
            pltpu.SemaphoreType.DMA((3, N_Z - 1)),
        ],
        compiler_params=pltpu.CompilerParams(
            collective_id=0,
            vmem_limit_bytes=100 * 1024 * 1024,
        ),
    )(x2, Wdkv, Wuk, Wuv)


G = 4
GDH = G * DH
GDR = G * DR


def _attention(x2, Wq, Wqr, Wkr, Wo, K, V):
    scale = (DH + DR) ** -0.5

    def body(x_ref, wq_ref, wqr_ref, wkr_ref, wo_ref, k_ref, v_ref, out_ref,
             kr_s):
        g = pl.program_id(0)
        xb = x_ref[...]
        wq4 = (wq_ref[...] * scale).astype(jnp.bfloat16)
        wqr4 = (wqr_ref[...] * scale).astype(jnp.bfloat16)
        wo4 = wo_ref[...].astype(jnp.bfloat16)
        q4 = jnp.dot(xb, wq4,
                     preferred_element_type=jnp.float32).astype(jnp.bfloat16)
        qr4 = jnp.dot(xb, wqr4,
                      preferred_element_type=jnp.float32).astype(jnp.bfloat16)

        @pl.when(g == 0)
        def _():
            wkrb = wkr_ref[...].astype(jnp.bfloat16)
            kr_s[...] = jnp.dot(
                xb, wkrb, preferred_element_type=jnp.float32
            ).astype(jnp.bfloat16)

        kr = kr_s[...]
        k4 = k_ref[...]
        v4 = v_ref[...]

        o_parts = []
        for i in range(G):
            qaug = jnp.concatenate(
                [q4[:, i * DH:(i + 1) * DH], qr4[:, i * DR:(i + 1) * DR]],
                axis=1)
            kaug = jnp.concatenate(
                [k4[:, i * DH:(i + 1) * DH], kr], axis=1)
            s = lax.dot_general(
                qaug, kaug, (((1,), (1,)), ((), ())),
                preferred_element_type=jnp.float32,
            )
            p = jnp.exp(s.astype(jnp.bfloat16))
            vh_ext = jnp.concatenate(
                [v4[:, i * DH:(i + 1) * DH],
                 jnp.ones((S, 1), jnp.bfloat16)], axis=1)
            oe = jnp.dot(p, vh_ext, preferred_element_type=jnp.float32)
            r = 1.0 / oe[:, DH:DH + 1]
            o_parts.append((oe[:, :DH] * r).astype(jnp.bfloat16))
        o4 = jnp.concatenate(o_parts, axis=1)
        contrib = jnp.dot(o4, wo4, preferred_element_type=jnp.float32)

        @pl.when(g == 0)
        def _():
            out_ref[...] = contrib

        @pl.when(g != 0)
        def _():
            out_ref[...] += contrib

    return pl.pallas_call(
        body,
        grid=(H // G,),
        out_shape=jax.ShapeDtypeStruct((S, D), jnp.float32),
        in_specs=[
            pl.BlockSpec((S, D), lambda g: (0, 0)),
            pl.BlockSpec((D, GDH), lambda g: (0, g)),
            pl.BlockSpec((D, GDR), lambda g: (0, g)),
            pl.BlockSpec((D, DR), lambda g: (0, 0)),
            pl.BlockSpec((GDH, D), lambda g: (g, 0)),
            pl.BlockSpec((S, GDH), lambda g: (0, g)),
            pl.BlockSpec((S, GDH), lambda g: (0, g)),
        ],
        out_specs=pl.BlockSpec((S, D), lambda g: (0, 0)),
        scratch_shapes=[
            pltpu.VMEM((S, DR), jnp.bfloat16),
        ],
        compiler_params=pltpu.CompilerParams(
            vmem_limit_bytes=100 * 1024 * 1024,
        ),
    )(x2, Wq, Wqr, Wkr, Wo, K, V)


def kernel(x, Wdkv, Wuk, Wuv, Wq, Wqr, Wkr, Wo):
    x2 = x[0].astype(jnp.bfloat16)
    K, V = _gather_kv(x[0], Wdkv, Wuk, Wuv)
    out = _attention(x2, Wq, Wqr, Wkr, Wo, K, V)
    return out[None]
